# baseline (speedup 1.0000x reference)
import jax
import jax.numpy as jnp
from jax import lax
from jax.experimental import pallas as pl
from jax.experimental.pallas import tpu as pltpu
from jax.experimental.pallas import tpu_sc as plsc

B = 16384
N_CAT = 26
N_CONT = 16
N_TOK = N_CAT + N_CONT
CARD = 100000
DIM = 32

NC = 2
NS = 16
NW = NC * NS

B_W = B // NW
CB = 16
N_CHUNKS = B_W // CB
R_CAT = CB * N_CAT
WIDE_OUT = CB * N_TOK * DIM // 128


def _sc_body(gidx4_hbm, qoff_hbm, xc_hbm, wb_hbm, table4_hbm,
             out_hbm,
             idx_v, qoff_v, wide_v, stage_v, xv, wbv, gsem):
    wid = lax.axis_index("s") * NC + lax.axis_index("c")

    pltpu.sync_copy(wb_hbm, wbv)

    def chunk(c, carry):
        g = wid * N_CHUNKS + c
        r0 = g * R_CAT
        pltpu.sync_copy(gidx4_hbm.at[pl.ds(r0, R_CAT)], idx_v)
        pltpu.sync_copy(qoff_hbm.at[pl.ds(r0, R_CAT)], qoff_v)
        pltpu.sync_copy(xc_hbm.at[pl.ds(g * CB * N_CONT, CB * N_CONT)], xv)
        pltpu.async_copy(table4_hbm.at[idx_v], wide_v, gsem).wait()

        for grp in range(R_CAT // 16):
            qv = qoff_v[pl.ds(grp * 16, 16)]
            for j in range(16):
                i = grp * 16 + j
                q = qv[j]
                dpos = (i // N_CAT) * N_TOK + (i % N_CAT)
                dr, dc = dpos >> 2, (dpos & 3) * DIM
                stage_v[dr, pl.ds(dc, 16)] = wide_v[i, pl.ds(q, 16)]
                stage_v[dr, pl.ds(dc + 16, 16)] = wide_v[i, pl.ds(q + 16, 16)]

        for j in range(CB):
            xrow = xv[pl.ds(j * N_CONT, N_CONT)]
            for f in range(N_CONT):
                xs = xrow[f]
                w0 = wbv[pl.ds(f * DIM, 16)]
                w1 = wbv[pl.ds(f * DIM + 16, 16)]
                b0 = wbv[pl.ds((N_CONT + f) * DIM, 16)]
                b1 = wbv[pl.ds((N_CONT + f) * DIM + 16, 16)]
                dpos = j * N_TOK + N_CAT + f
                dr, dc = dpos >> 2, (dpos & 3) * DIM
                stage_v[dr, pl.ds(dc, 16)] = xs * w0 + b0
                stage_v[dr, pl.ds(dc + 16, 16)] = xs * w1 + b1

        pltpu.sync_copy(stage_v, out_hbm.at[pl.ds(g * WIDE_OUT, WIDE_OUT)])
        return carry

    lax.fori_loop(0, N_CHUNKS, chunk, 0)


@jax.jit
def kernel(x_cat, x_cont, cat_table, cont_W, cont_b):
    offsets = jnp.arange(N_CAT, dtype=jnp.int32) * CARD
    flat = (x_cat.astype(jnp.int32) + offsets[None, :]).reshape(-1)
    gidx4 = flat >> 2
    qoff = (flat & 3) * DIM
    table4 = cat_table.reshape(N_CAT * CARD // 4, 128)
    xc = x_cont.reshape(-1)
    wb = jnp.concatenate([cont_W.reshape(-1), cont_b.reshape(-1)])

    mesh = plsc.VectorSubcoreMesh(core_axis_name="c", subcore_axis_name="s",
                                  num_cores=NC, num_subcores=NS)
    out = pl.kernel(
        _sc_body,
        out_type=jax.ShapeDtypeStruct((B * N_TOK * DIM // 128, 128),
                                      jnp.float32),
        mesh=mesh,
        scratch_types=[
            pltpu.VMEM((R_CAT,), jnp.int32),
            pltpu.VMEM((R_CAT,), jnp.int32),
            pltpu.VMEM((R_CAT, 128), jnp.float32),
            pltpu.VMEM((WIDE_OUT, 128), jnp.float32),
            pltpu.VMEM((CB * N_CONT,), jnp.float32),
            pltpu.VMEM((2 * N_CONT * DIM,), jnp.float32),
            pltpu.SemaphoreType.DMA,
        ],
        compiler_params=pltpu.CompilerParams(use_tc_tiling_on_sc=True),
    )(gidx4, qoff, xc, wb, table4)
    return out.reshape(B, N_TOK, DIM)
